# baseline (speedup 1.0000x reference)
import functools

import jax
import jax.numpy as jnp
from jax import lax
from jax.experimental import pallas as pl
from jax.experimental.pallas import tpu as pltpu
from jax.experimental.pallas import tpu_sc as plsc

SEQ_LEN = 8192
D = 768
B = 4
EPS = 1e-12

BLOCK_ROWS = 2048


def _fused_ln_kernel(x_ref, p_ref, g_ref, b_ref, o_ref):
    x = x_ref[...] + p_ref[...]
    mean = jnp.mean(x, axis=-1, keepdims=True)
    xc = x - mean
    var = jnp.mean(xc * xc, axis=-1, keepdims=True)
    o_ref[...] = xc * jax.lax.rsqrt(var + EPS) * g_ref[...] + b_ref[...]


def _tc_kernel(inputs_embeds, pos_table, ln_gamma, ln_beta):
    b, s, d = inputs_embeds.shape
    g = ln_gamma.reshape(1, d)
    bt = ln_beta.reshape(1, d)
    grid = (s // BLOCK_ROWS, b)
    return pl.pallas_call(
        _fused_ln_kernel,
        grid=grid,
        in_specs=[
            pl.BlockSpec((1, BLOCK_ROWS, d), lambda i, j: (j, i, 0)),
            pl.BlockSpec((BLOCK_ROWS, d), lambda i, j: (i, 0)),
            pl.BlockSpec((1, d), lambda i, j: (0, 0)),
            pl.BlockSpec((1, d), lambda i, j: (0, 0)),
        ],
        out_specs=pl.BlockSpec((1, BLOCK_ROWS, d), lambda i, j: (j, i, 0)),
        out_shape=jax.ShapeDtypeStruct((b, s, d), inputs_embeds.dtype),
        compiler_params=pltpu.CompilerParams(
            dimension_semantics=("parallel", "parallel"),
        ),
    )(inputs_embeds, pos_table, g, bt)


NC = 2
NS = 16
L = 16
NW = NC * NS
ROWS_PER_W = SEQ_LEN // NW
CH = 32
NCH = ROWS_PER_W // CH
NJ = D // L


def _lane_sum(x):
    iota = lax.iota(jnp.int32, L)
    for k in (8, 4, 2, 1):
        perm = jnp.bitwise_xor(iota, k)
        x = x + jnp.take(x, perm)
    return x


def _newton_rsqrt(v):
    bits = lax.bitcast_convert_type(v, jnp.int32)
    y = lax.bitcast_convert_type(jnp.int32(0x5F3759DF) - (bits >> 1), jnp.float32)
    for _ in range(3):
        y = y * (1.5 - 0.5 * v * y * y)
    return y


def _sc_body(x_hbm, p_hbm, g_hbm, b_hbm, o_hbm, x_v, p_v, o_v, g_v, b_v):
    wid = lax.axis_index("s") * NC + lax.axis_index("c")
    base = wid * ROWS_PER_W
    pltpu.sync_copy(g_hbm, g_v)
    pltpu.sync_copy(b_hbm, b_v)

    def chunk_body(c, carry):
        row0 = base + c * CH
        pltpu.sync_copy(p_hbm.at[pl.ds(row0, CH), :], p_v)
        for bb in range(B):
            pltpu.sync_copy(x_hbm.at[bb, pl.ds(row0, CH), :], x_v)

            @plsc.parallel_loop(0, CH, unroll=2)
            def _rows(r):
                z = jnp.zeros((L,), jnp.float32)
                s_acc, q_acc = z, z
                for j in range(NJ):
                    sl = pl.ds(j * L, L)
                    w = x_v[r, sl] + p_v[r, sl]
                    o_v[r, sl] = w
                    s_acc = s_acc + w
                    q_acc = q_acc + w * w
                meanv = _lane_sum(s_acc) * (1.0 / D)
                msqv = _lane_sum(q_acc) * (1.0 / D)
                varv = msqv - meanv * meanv + EPS
                rs = _newton_rsqrt(varv)
                for j in range(NJ):
                    sl = pl.ds(j * L, L)
                    w = o_v[r, sl]
                    o_v[r, sl] = (w - meanv) * rs * g_v[sl] + b_v[sl]
            pltpu.sync_copy(o_v, o_hbm.at[bb, pl.ds(row0, CH), :])
        return carry

    lax.fori_loop(0, NCH, chunk_body, 0)


def _sc_kernel(inputs_embeds, pos_table, ln_gamma, ln_beta):
    mesh = plsc.VectorSubcoreMesh(core_axis_name="c", subcore_axis_name="s")
    fn = pl.kernel(
        _sc_body,
        out_type=jax.ShapeDtypeStruct((B, SEQ_LEN, D), jnp.float32),
        mesh=mesh,
        scratch_types=[
            pltpu.VMEM((CH, D), jnp.float32),
            pltpu.VMEM((CH, D), jnp.float32),
            pltpu.VMEM((CH, D), jnp.float32),
            pltpu.VMEM((D,), jnp.float32),
            pltpu.VMEM((D,), jnp.float32),
        ],
    )
    return fn(inputs_embeds, pos_table, ln_gamma, ln_beta)


def kernel(inputs_embeds, pos_table, ln_gamma, ln_beta):
    return _sc_kernel(inputs_embeds, pos_table, ln_gamma, ln_beta)


# speedup vs baseline: 5.1517x; 5.1517x over previous
import functools

import jax
import jax.numpy as jnp
from jax import lax
from jax.experimental import pallas as pl
from jax.experimental.pallas import tpu as pltpu
from jax.experimental.pallas import tpu_sc as plsc

SEQ_LEN = 8192
D = 768
B = 4
EPS = 1e-12

BLOCK_ROWS = 2048


def _fused_ln_kernel(x_ref, p_ref, g_ref, b_ref, o_ref):
    x = x_ref[...] + p_ref[...]
    mean = jnp.mean(x, axis=-1, keepdims=True)
    xc = x - mean
    var = jnp.mean(xc * xc, axis=-1, keepdims=True)
    o_ref[...] = xc * jax.lax.rsqrt(var + EPS) * g_ref[...] + b_ref[...]


def _tc_kernel(inputs_embeds, pos_table, ln_gamma, ln_beta):
    b, s, d = inputs_embeds.shape
    g = ln_gamma.reshape(1, d)
    bt = ln_beta.reshape(1, d)
    grid = (s // BLOCK_ROWS, b)
    return pl.pallas_call(
        _fused_ln_kernel,
        grid=grid,
        in_specs=[
            pl.BlockSpec((1, BLOCK_ROWS, d), lambda i, j: (j, i, 0)),
            pl.BlockSpec((BLOCK_ROWS, d), lambda i, j: (i, 0)),
            pl.BlockSpec((1, d), lambda i, j: (0, 0)),
            pl.BlockSpec((1, d), lambda i, j: (0, 0)),
        ],
        out_specs=pl.BlockSpec((1, BLOCK_ROWS, d), lambda i, j: (j, i, 0)),
        out_shape=jax.ShapeDtypeStruct((b, s, d), inputs_embeds.dtype),
        compiler_params=pltpu.CompilerParams(
            dimension_semantics=("parallel", "parallel"),
        ),
    )(inputs_embeds, pos_table, g, bt)


NC = 2
NS = 16
L = 16
NW = NC * NS
ROWS_PER_W = SEQ_LEN // NW
CH = 32
NCH = ROWS_PER_W // CH
NJ = D // L


def _lane_sum(x):
    iota = lax.iota(jnp.int32, L)
    for k in (8, 4, 2, 1):
        perm = jnp.bitwise_xor(iota, k)
        x = x + jnp.take(x, perm)
    return x


def _newton_rsqrt(v):
    bits = lax.bitcast_convert_type(v, jnp.int32)
    y = lax.bitcast_convert_type(jnp.int32(0x5F3759DF) - (bits >> 1), jnp.float32)
    for _ in range(3):
        y = y * (1.5 - 0.5 * v * y * y)
    return y


def _sc_body(x_hbm, p_hbm, g_hbm, b_hbm, o_hbm, x_v, p_v, o_v, g_v, b_v):
    wid = lax.axis_index("s") * NC + lax.axis_index("c")
    base = wid * ROWS_PER_W
    pltpu.sync_copy(g_hbm, g_v)
    pltpu.sync_copy(b_hbm, b_v)

    def chunk_body(c, carry):
        row0 = base + c * CH
        pltpu.sync_copy(p_hbm.at[pl.ds(row0, CH), :], p_v)
        for bb in range(B):
            pltpu.sync_copy(x_hbm.at[bb, pl.ds(row0, CH), :], x_v)

            @plsc.parallel_loop(0, CH, unroll=2)
            def _rows(r):
                z = jnp.zeros((L,), jnp.float32)
                s_acc, q_acc = z, z
                for j in range(NJ):
                    sl = pl.ds(j * L, L)
                    w = x_v[r, sl] + p_v[r, sl]
                    o_v[r, sl] = w
                    s_acc = s_acc + w
                    q_acc = q_acc + w * w
                meanv = _lane_sum(s_acc) * (1.0 / D)
                msqv = _lane_sum(q_acc) * (1.0 / D)
                varv = msqv - meanv * meanv + EPS
                rs = _newton_rsqrt(varv)
                for j in range(NJ):
                    sl = pl.ds(j * L, L)
                    w = o_v[r, sl]
                    o_v[r, sl] = (w - meanv) * rs * g_v[sl] + b_v[sl]
            pltpu.sync_copy(o_v, o_hbm.at[bb, pl.ds(row0, CH), :])
        return carry

    lax.fori_loop(0, NCH, chunk_body, 0)


def _sc_kernel(inputs_embeds, pos_table, ln_gamma, ln_beta):
    mesh = plsc.VectorSubcoreMesh(core_axis_name="c", subcore_axis_name="s")
    fn = pl.kernel(
        _sc_body,
        out_type=jax.ShapeDtypeStruct((B, SEQ_LEN, D), jnp.float32),
        mesh=mesh,
        scratch_types=[
            pltpu.VMEM((CH, D), jnp.float32),
            pltpu.VMEM((CH, D), jnp.float32),
            pltpu.VMEM((CH, D), jnp.float32),
            pltpu.VMEM((D,), jnp.float32),
            pltpu.VMEM((D,), jnp.float32),
        ],
    )
    return fn(inputs_embeds, pos_table, ln_gamma, ln_beta)


def kernel(inputs_embeds, pos_table, ln_gamma, ln_beta):
    return _tc_kernel(inputs_embeds, pos_table, ln_gamma, ln_beta)
